# baseline (speedup 1.0000x reference)
import jax
import jax.numpy as jnp
import numpy as np
from jax.experimental import pallas as pl
from jax.experimental.pallas import tpu as pltpu

_IDX = (3, 7, 15, 22, 31, 44, 58, 63, 71, 85, 92, 101, 110, 118, 124, 127)

_SEL = np.zeros((128, 16), dtype=np.float32)
for _k, _i in enumerate(_IDX):
    _SEL[_i, _k] = 1.0


_NB = 2048


def _gather_body(x_ref, s_ref, o_ref):
    s = s_ref[...]
    for fj in range(8):
        xf = x_ref[:, 0, fj, :]
        g = jax.lax.dot(xf, s, precision=jax.lax.Precision.HIGHEST,
                        preferred_element_type=jnp.float32)
        o_ref[fj, :, :] = g.T


def kernel(inputs):
    n = inputs.shape[0]
    sel = jnp.asarray(_SEL)
    x4 = inputs.reshape(n, 25, 8, 128)
    out_t = pl.pallas_call(
        _gather_body,
        grid=(25, n // _NB),
        in_specs=[
            pl.BlockSpec((_NB, 1, 8, 128), lambda f, b: (b, f, 0, 0)),
            pl.BlockSpec((128, 16), lambda f, b: (0, 0)),
        ],
        out_specs=pl.BlockSpec((8, 16, _NB), lambda f, b: (f, 0, b)),
        out_shape=jax.ShapeDtypeStruct((200, 16, n), inputs.dtype),
        compiler_params=pltpu.CompilerParams(
            dimension_semantics=("parallel", "parallel")),
    )(x4, sel)
    return out_t.transpose(2, 0, 1)


# speedup vs baseline: 1.4500x; 1.4500x over previous
import jax
import jax.numpy as jnp
import numpy as np
from jax.experimental import pallas as pl
from jax.experimental.pallas import tpu as pltpu

_IDX = (3, 7, 15, 22, 31, 44, 58, 63, 71, 85, 92, 101, 110, 118, 124, 127)

_SEL_T = np.zeros((16, 128), dtype=np.float32)
for _k, _i in enumerate(_IDX):
    _SEL_T[_k, _i] = 1.0

_NB = 2048


def _gather_body(x_ref, s_ref, o_ref):
    x = x_ref[...].reshape(_NB, 128)
    g_t = jax.lax.dot_general(
        s_ref[...], x, (((1,), (1,)), ((), ())),
        preferred_element_type=jnp.float32)
    o_ref[...] = g_t.reshape(1, 16, _NB)


def kernel(inputs):
    n = inputs.shape[0]
    sel_t = jnp.asarray(_SEL_T)
    x4 = inputs.reshape(n, 200, 1, 128)
    out_t = pl.pallas_call(
        _gather_body,
        grid=(200, n // _NB),
        in_specs=[
            pl.BlockSpec((_NB, 1, 1, 128), lambda f, b: (b, f, 0, 0)),
            pl.BlockSpec((16, 128), lambda f, b: (0, 0)),
        ],
        out_specs=pl.BlockSpec((1, 16, _NB), lambda f, b: (f, 0, b)),
        out_shape=jax.ShapeDtypeStruct((200, 16, n), inputs.dtype),
        compiler_params=pltpu.CompilerParams(
            dimension_semantics=("parallel", "parallel")),
    )(x4, sel_t)
    return out_t.transpose(2, 0, 1)


# speedup vs baseline: 1.9972x; 1.3773x over previous
import jax
import jax.numpy as jnp
import numpy as np
from jax.experimental import pallas as pl
from jax.experimental.pallas import tpu as pltpu

_IDX = (3, 7, 15, 22, 31, 44, 58, 63, 71, 85, 92, 101, 110, 118, 124, 127)

_SEL_T = np.zeros((16, 128), dtype=np.float32)
for _k, _i in enumerate(_IDX):
    _SEL_T[_k, _i] = 1.0

_NB = 4096


def _gather_body(x_ref, s_ref, o_ref):
    x = x_ref[...].reshape(_NB, 128)
    g_t = jax.lax.dot_general(
        s_ref[...], x, (((1,), (1,)), ((), ())),
        preferred_element_type=jnp.float32)
    o_ref[...] = g_t.reshape(1, 16, _NB)


def kernel(inputs):
    n = inputs.shape[0]
    sel_t = jnp.asarray(_SEL_T)
    x4 = inputs.reshape(n, 200, 1, 128)
    out_t = pl.pallas_call(
        _gather_body,
        grid=(n // _NB, 200),
        in_specs=[
            pl.BlockSpec((_NB, 1, 1, 128), lambda b, f: (b, f, 0, 0)),
            pl.BlockSpec((16, 128), lambda b, f: (0, 0)),
        ],
        out_specs=pl.BlockSpec((1, 16, _NB), lambda b, f: (f, 0, b)),
        out_shape=jax.ShapeDtypeStruct((200, 16, n), inputs.dtype),
        compiler_params=pltpu.CompilerParams(
            dimension_semantics=("parallel", "parallel")),
    )(x4, sel_t)
    return out_t.transpose(2, 0, 1)


# speedup vs baseline: 2.5295x; 1.2665x over previous
import jax
import jax.numpy as jnp
import numpy as np
from jax.experimental import pallas as pl
from jax.experimental.pallas import tpu as pltpu

_IDX = (3, 7, 15, 22, 31, 44, 58, 63, 71, 85, 92, 101, 110, 118, 124, 127)

_SEL_T = np.zeros((16, 128), dtype=np.float32)
for _k, _i in enumerate(_IDX):
    _SEL_T[_k, _i] = 1.0

_NF = 200


def _gather_body(x_hbm, s_ref, o_ref, xs_ref, sem):
    f = pl.program_id(0)

    @pl.when(f == 0)
    def _first():
        pltpu.make_async_copy(x_hbm.at[:, 0, :], xs_ref.at[0], sem.at[0]).start()

    @pl.when(f + 1 < _NF)
    def _prefetch():
        pltpu.make_async_copy(
            x_hbm.at[:, f + 1, :], xs_ref.at[(f + 1) % 2], sem.at[(f + 1) % 2]
        ).start()

    pltpu.make_async_copy(x_hbm.at[:, f, :], xs_ref.at[f % 2], sem.at[f % 2]).wait()
    x = xs_ref[f % 2]
    g_t = jax.lax.dot_general(
        s_ref[...], x, (((1,), (1,)), ((), ())),
        preferred_element_type=jnp.float32)
    o_ref[...] = g_t.reshape(o_ref.shape)


def kernel(inputs):
    n = inputs.shape[0]
    sel_t = jnp.asarray(_SEL_T)
    out_t = pl.pallas_call(
        _gather_body,
        grid=(_NF,),
        in_specs=[
            pl.BlockSpec(memory_space=pl.ANY),
            pl.BlockSpec((16, 128), lambda f: (0, 0)),
        ],
        out_specs=pl.BlockSpec((1, 16, n), lambda f: (f, 0, 0)),
        out_shape=jax.ShapeDtypeStruct((200, 16, n), inputs.dtype),
        scratch_shapes=[
            pltpu.VMEM((2, n, 128), jnp.float32),
            pltpu.SemaphoreType.DMA((2,)),
        ],
        compiler_params=pltpu.CompilerParams(
            dimension_semantics=("arbitrary",)),
    )(inputs, sel_t)
    return out_t.transpose(2, 0, 1)


# speedup vs baseline: 4.0187x; 1.5887x over previous
import jax
import jax.numpy as jnp
import numpy as np
from jax.experimental import pallas as pl
from jax.experimental.pallas import tpu as pltpu

_IDX = (3, 7, 15, 22, 31, 44, 58, 63, 71, 85, 92, 101, 110, 118, 124, 127)

_SEL_T = np.zeros((16, 128), dtype=np.float32)
for _k, _i in enumerate(_IDX):
    _SEL_T[_k, _i] = 1.0

_NF = 200


_DEPTH = 4


def _gather_body(x_hbm, s_ref, o_ref, xs_ref, sem):
    f = pl.program_id(0)

    @pl.when(f == 0)
    def _first():
        for d in range(_DEPTH - 1):
            pltpu.make_async_copy(x_hbm.at[:, d, :], xs_ref.at[d], sem.at[d]).start()

    @pl.when(f + _DEPTH - 1 < _NF)
    def _prefetch():
        nxt = f + _DEPTH - 1
        pltpu.make_async_copy(
            x_hbm.at[:, nxt, :], xs_ref.at[nxt % _DEPTH], sem.at[nxt % _DEPTH]
        ).start()

    pltpu.make_async_copy(
        x_hbm.at[:, f, :], xs_ref.at[f % _DEPTH], sem.at[f % _DEPTH]).wait()
    x = xs_ref[f % _DEPTH]
    g_t = jax.lax.dot_general(
        s_ref[...], x, (((1,), (1,)), ((), ())),
        preferred_element_type=jnp.float32)
    o_ref[...] = g_t.reshape(o_ref.shape)


def kernel(inputs):
    n = inputs.shape[0]
    sel_t = jnp.asarray(_SEL_T)
    out_t = pl.pallas_call(
        _gather_body,
        grid=(_NF,),
        in_specs=[
            pl.BlockSpec(memory_space=pl.ANY),
            pl.BlockSpec((16, 128), lambda f: (0, 0)),
        ],
        out_specs=pl.BlockSpec((1, 16, n), lambda f: (f, 0, 0)),
        out_shape=jax.ShapeDtypeStruct((200, 16, n), inputs.dtype),
        scratch_shapes=[
            pltpu.VMEM((_DEPTH, n, 128), jnp.float32),
            pltpu.SemaphoreType.DMA((_DEPTH,)),
        ],
        compiler_params=pltpu.CompilerParams(
            dimension_semantics=("arbitrary",)),
    )(inputs, sel_t)
    return out_t.transpose(2, 0, 1)


# speedup vs baseline: 4.0934x; 1.0186x over previous
import jax
import jax.numpy as jnp
import numpy as np
from jax.experimental import pallas as pl
from jax.experimental.pallas import tpu as pltpu

_IDX = (3, 7, 15, 22, 31, 44, 58, 63, 71, 85, 92, 101, 110, 118, 124, 127)

_SEL_T = np.zeros((16, 128), dtype=np.float32)
for _k, _i in enumerate(_IDX):
    _SEL_T[_k, _i] = 1.0

_NF = 200


_DEPTH = 8


def _gather_body(x_hbm, s_ref, o_ref, xs_ref, sem):
    f = pl.program_id(0)

    @pl.when(f == 0)
    def _first():
        for d in range(_DEPTH - 1):
            pltpu.make_async_copy(x_hbm.at[:, d, :], xs_ref.at[d], sem.at[d]).start()

    @pl.when(f + _DEPTH - 1 < _NF)
    def _prefetch():
        nxt = f + _DEPTH - 1
        pltpu.make_async_copy(
            x_hbm.at[:, nxt, :], xs_ref.at[nxt % _DEPTH], sem.at[nxt % _DEPTH]
        ).start()

    pltpu.make_async_copy(
        x_hbm.at[:, f, :], xs_ref.at[f % _DEPTH], sem.at[f % _DEPTH]).wait()
    x = xs_ref[f % _DEPTH]
    g_t = jax.lax.dot_general(
        s_ref[...], x, (((1,), (1,)), ((), ())),
        preferred_element_type=jnp.float32)
    o_ref[...] = g_t.reshape(o_ref.shape)


def kernel(inputs):
    n = inputs.shape[0]
    sel_t = jnp.asarray(_SEL_T)
    out_t = pl.pallas_call(
        _gather_body,
        grid=(_NF,),
        in_specs=[
            pl.BlockSpec(memory_space=pl.ANY),
            pl.BlockSpec((16, 128), lambda f: (0, 0)),
        ],
        out_specs=pl.BlockSpec((1, 16, n), lambda f: (f, 0, 0)),
        out_shape=jax.ShapeDtypeStruct((200, 16, n), inputs.dtype),
        scratch_shapes=[
            pltpu.VMEM((_DEPTH, n, 128), jnp.float32),
            pltpu.SemaphoreType.DMA((_DEPTH,)),
        ],
        compiler_params=pltpu.CompilerParams(
            dimension_semantics=("arbitrary",)),
    )(inputs, sel_t)
    return out_t.transpose(2, 0, 1)
